# Initial kernel scaffold; baseline (speedup 1.0000x reference)
import jax
import jax.numpy as jnp
from jax.experimental import pallas as pl


def kernel(particles, weights, edges):
    raise NotImplementedError("write your pallas kernel here")


# speedup vs baseline: 4.4983x; 4.4983x over previous
import functools

import jax
import jax.numpy as jnp
from jax import lax
from jax.experimental import pallas as pl
from jax.experimental.pallas import tpu as pltpu
from jax.experimental.pallas import tpu_sc as plsc

N_NODES = 10000
N_EDGES = 320000
D = 128
LANES = 16

NC, NS = 2, 16
NW = NC * NS
E_PER_W = N_EDGES // NW
CH = 80
NCHUNK = E_PER_W // CH

_mesh = plsc.VectorSubcoreMesh(core_axis_name="c", subcore_axis_name="s")


@functools.partial(
    pl.kernel,
    out_type=jax.ShapeDtypeStruct((N_EDGES, D), jnp.float32),
    mesh=_mesh,
    scratch_types=[
        pltpu.VMEM((E_PER_W,), jnp.int32),
        pltpu.VMEM((E_PER_W,), jnp.int32),
        pltpu.VMEM((CH, D), jnp.float32),
        pltpu.VMEM((CH, D), jnp.float32),
        pltpu.SemaphoreType.DMA,
        pltpu.SemaphoreType.DMA,
    ],
)
def _ratios_sc(table, idx_i, idx_j, out, ii_v, jj_v, ri_v, rj_v, sem_i, sem_j):
    wid = lax.axis_index("s") * NC + lax.axis_index("c")
    base = wid * E_PER_W
    pltpu.sync_copy(idx_i.at[pl.ds(base, E_PER_W)], ii_v)
    pltpu.sync_copy(idx_j.at[pl.ds(base, E_PER_W)], jj_v)

    def chunk_body(c, carry):
        off = c * CH
        ci = pltpu.async_copy(table.at[ii_v.at[pl.ds(off, CH)]], ri_v, sem_i)
        cj = pltpu.async_copy(table.at[jj_v.at[pl.ds(off, CH)]], rj_v, sem_j)
        ci.wait()
        cj.wait()

        def row_body(r, rcarry):
            for k in range(D // LANES):
                s = pl.ds(k * LANES, LANES)
                ri_v[r, s] = ri_v[r, s] - rj_v[r, s]
            return rcarry

        lax.fori_loop(0, CH, row_body, 0, unroll=False)
        pltpu.sync_copy(ri_v, out.at[pl.ds(base + off, CH)])
        return carry

    lax.fori_loop(0, NCHUNK, chunk_body, 0, unroll=False)


def _weights_tc_body(w_ref, o_ref):
    o_ref[...] = jnp.broadcast_to(w_ref[...], o_ref.shape)


_W_BLK = 3200


def _weights_tc(weights):
    return pl.pallas_call(
        _weights_tc_body,
        grid=(N_EDGES // _W_BLK,),
        in_specs=[pl.BlockSpec((1, D), lambda i: (0, 0))],
        out_specs=pl.BlockSpec((_W_BLK, D), lambda i: (i, 0)),
        out_shape=jax.ShapeDtypeStruct((N_EDGES, D), jnp.float32),
    )(weights)


def kernel(particles, weights, edges):
    table = particles.reshape(N_NODES, D)
    idx = edges.astype(jnp.int32)
    idx_i = idx[:, 0]
    idx_j = idx[:, 1]
    ratios = _ratios_sc(table, idx_i, idx_j)
    rm_weights = _weights_tc(weights)
    return ratios.reshape(N_EDGES, D, 1), rm_weights
